# baseline (speedup 1.0000x reference)
import jax
import jax.numpy as jnp
from jax.experimental import pallas as pl
from jax.experimental.pallas import tpu as pltpu

NUM_CLASSES = 10000
CLASSES_PER_TASK = 1000
CURRENT_TASK = 1
BAND_START = CURRENT_TASK * CLASSES_PER_TASK
BAND_END = BAND_START + CLASSES_PER_TASK

HULL_START = 768
HULL_WIDTH = 1280

ROWS = 4096
ROW_CHUNK = 512
N_CHUNKS = ROWS // ROW_CHUNK


def _body(alpha_ref, beta_ref, x_hbm, o_hbm, in_buf, out_buf,
          in_sem, out_sem, side_sem):
    def side_copy(i):
        return pltpu.make_async_copy(
            x_hbm.at[pl.ds(i * ROW_CHUNK, ROW_CHUNK), :],
            o_hbm.at[pl.ds(i * ROW_CHUNK, ROW_CHUNK), :],
            side_sem.at[i],
        )

    def band_in(i, slot):
        return pltpu.make_async_copy(
            x_hbm.at[pl.ds(i * ROW_CHUNK, ROW_CHUNK),
                     pl.ds(HULL_START, HULL_WIDTH)],
            in_buf.at[slot],
            in_sem.at[slot],
        )

    def band_out(i, slot):
        return pltpu.make_async_copy(
            out_buf.at[slot],
            o_hbm.at[pl.ds(i * ROW_CHUNK, ROW_CHUNK),
                     pl.ds(HULL_START, HULL_WIDTH)],
            out_sem.at[slot],
        )

    for i in range(N_CHUNKS):
        side_copy(i).start()

    a = alpha_ref[0]
    b = beta_ref[0]
    col = HULL_START + jax.lax.broadcasted_iota(
        jnp.int32, (ROW_CHUNK, HULL_WIDTH), dimension=1)
    in_band = (col >= BAND_START) & (col < BAND_END)

    band_in(0, 0).start()

    def step(i, _):
        slot = jax.lax.rem(i, 2)
        nslot = jax.lax.rem(i + 1, 2)

        @pl.when(i + 1 < N_CHUNKS)
        def _():
            band_in(i + 1, nslot).start()

        band_in(i, slot).wait()

        @pl.when(i >= 2)
        def _():
            band_out(i - 2, slot).wait()

        xv = in_buf[slot]
        out_buf[slot] = jnp.where(in_band, xv * a + b, xv)
        side_copy(i).wait()
        band_out(i, slot).start()
        return 0

    jax.lax.fori_loop(0, N_CHUNKS, step, 0)

    @pl.when(N_CHUNKS >= 2)
    def _():
        band_out(N_CHUNKS - 2, jax.lax.rem(N_CHUNKS - 2, 2)).wait()

    band_out(N_CHUNKS - 1, jax.lax.rem(N_CHUNKS - 1, 2)).wait()


def kernel(x, alpha, beta):
    m, n = x.shape
    return pl.pallas_call(
        _body,
        in_specs=[
            pl.BlockSpec(memory_space=pltpu.SMEM),
            pl.BlockSpec(memory_space=pltpu.SMEM),
            pl.BlockSpec(memory_space=pltpu.HBM),
        ],
        out_specs=pl.BlockSpec(memory_space=pltpu.HBM),
        out_shape=jax.ShapeDtypeStruct((m, n), x.dtype),
        scratch_shapes=[
            pltpu.VMEM((2, ROW_CHUNK, HULL_WIDTH), jnp.float32),
            pltpu.VMEM((2, ROW_CHUNK, HULL_WIDTH), jnp.float32),
            pltpu.SemaphoreType.DMA((2,)),
            pltpu.SemaphoreType.DMA((2,)),
            pltpu.SemaphoreType.DMA((N_CHUNKS,)),
        ],
    )(alpha, beta, x)


# speedup vs baseline: 13.3162x; 13.3162x over previous
import jax
import jax.numpy as jnp
from jax.experimental import pallas as pl
from jax.experimental.pallas import tpu as pltpu

NUM_CLASSES = 10000
CLASSES_PER_TASK = 1000
CURRENT_TASK = 1
BAND_START = CURRENT_TASK * CLASSES_PER_TASK
BAND_END = BAND_START + CLASSES_PER_TASK

ROWS = 4096
ROW_CHUNK = 128
N_CHUNKS = ROWS // ROW_CHUNK
SLOTS = 5


def _body(alpha_ref, beta_ref, x_hbm, o_hbm, in_buf, out_buf, in_sem, out_sem):
    def chunk_in(i, slot):
        return pltpu.make_async_copy(
            x_hbm.at[pl.ds(i * ROW_CHUNK, ROW_CHUNK), :],
            in_buf.at[slot],
            in_sem.at[slot],
        )

    def chunk_out(i, slot):
        return pltpu.make_async_copy(
            out_buf.at[slot],
            o_hbm.at[pl.ds(i * ROW_CHUNK, ROW_CHUNK), :],
            out_sem.at[slot],
        )

    for i in range(SLOTS):
        chunk_in(i, i).start()

    a = alpha_ref[0]
    b = beta_ref[0]
    col = jax.lax.broadcasted_iota(
        jnp.int32, (ROW_CHUNK, NUM_CLASSES), dimension=1)
    in_band = (col >= BAND_START) & (col < BAND_END)

    def step(i, _):
        slot = jax.lax.rem(i, SLOTS)
        chunk_in(i, slot).wait()

        @pl.when(i >= SLOTS)
        def _():
            chunk_out(i - SLOTS, slot).wait()

        xv = in_buf[slot]
        out_buf[slot] = jnp.where(in_band, xv * a + b, xv)
        chunk_out(i, slot).start()

        @pl.when(i + SLOTS < N_CHUNKS)
        def _():
            chunk_in(i + SLOTS, slot).start()

        return 0

    jax.lax.fori_loop(0, N_CHUNKS, step, 0)

    for i in range(N_CHUNKS - SLOTS, N_CHUNKS):
        chunk_out(i, i % SLOTS).wait()


def kernel(x, alpha, beta):
    m, n = x.shape
    return pl.pallas_call(
        _body,
        in_specs=[
            pl.BlockSpec(memory_space=pltpu.SMEM),
            pl.BlockSpec(memory_space=pltpu.SMEM),
            pl.BlockSpec(memory_space=pltpu.HBM),
        ],
        out_specs=pl.BlockSpec(memory_space=pltpu.HBM),
        out_shape=jax.ShapeDtypeStruct((m, n), x.dtype),
        scratch_shapes=[
            pltpu.VMEM((SLOTS, ROW_CHUNK, NUM_CLASSES), jnp.float32),
            pltpu.VMEM((SLOTS, ROW_CHUNK, NUM_CLASSES), jnp.float32),
            pltpu.SemaphoreType.DMA((SLOTS,)),
            pltpu.SemaphoreType.DMA((SLOTS,)),
        ],
    )(alpha, beta, x)


# speedup vs baseline: 13.3800x; 1.0048x over previous
import jax
import jax.numpy as jnp
from jax.experimental import pallas as pl
from jax.experimental.pallas import tpu as pltpu

NUM_CLASSES = 10000
CLASSES_PER_TASK = 1000
CURRENT_TASK = 1
BAND_START = CURRENT_TASK * CLASSES_PER_TASK
BAND_END = BAND_START + CLASSES_PER_TASK

ROWS = 4096
ROW_CHUNK = 64
Q = 4
DEPTH = 2
STEP_ROWS = ROW_CHUNK * Q
N_STEPS = ROWS // STEP_ROWS


def _body(alpha_ref, beta_ref, x_hbm, o_hbm, in_buf, out_buf, in_sem, out_sem):
    def chunk_in(q, i, slot):
        return pltpu.make_async_copy(
            x_hbm.at[pl.ds((i * Q + q) * ROW_CHUNK, ROW_CHUNK), :],
            in_buf.at[q, slot],
            in_sem.at[q, slot],
        )

    def chunk_out(q, i, slot):
        return pltpu.make_async_copy(
            out_buf.at[q, slot],
            o_hbm.at[pl.ds((i * Q + q) * ROW_CHUNK, ROW_CHUNK), :],
            out_sem.at[q, slot],
        )

    for q in range(Q):
        for d in range(DEPTH):
            chunk_in(q, d, d).start()

    a = alpha_ref[0]
    b = beta_ref[0]
    col = jax.lax.broadcasted_iota(
        jnp.int32, (ROW_CHUNK, NUM_CLASSES), dimension=1)
    in_band = (col >= BAND_START) & (col < BAND_END)

    def step(i, _):
        slot = jax.lax.rem(i, DEPTH)
        for q in range(Q):
            chunk_in(q, i, slot).wait()

            @pl.when(i >= DEPTH)
            def _():
                chunk_out(q, i - DEPTH, slot).wait()

            xv = in_buf[q, slot]
            out_buf[q, slot] = jnp.where(in_band, xv * a + b, xv)
            chunk_out(q, i, slot).start()

            @pl.when(i + DEPTH < N_STEPS)
            def _():
                chunk_in(q, i + DEPTH, slot).start()

        return 0

    jax.lax.fori_loop(0, N_STEPS, step, 0)

    for q in range(Q):
        for i in range(N_STEPS - DEPTH, N_STEPS):
            chunk_out(q, i, i % DEPTH).wait()


def kernel(x, alpha, beta):
    m, n = x.shape
    return pl.pallas_call(
        _body,
        in_specs=[
            pl.BlockSpec(memory_space=pltpu.SMEM),
            pl.BlockSpec(memory_space=pltpu.SMEM),
            pl.BlockSpec(memory_space=pltpu.HBM),
        ],
        out_specs=pl.BlockSpec(memory_space=pltpu.HBM),
        out_shape=jax.ShapeDtypeStruct((m, n), x.dtype),
        scratch_shapes=[
            pltpu.VMEM((Q, DEPTH, ROW_CHUNK, NUM_CLASSES), jnp.float32),
            pltpu.VMEM((Q, DEPTH, ROW_CHUNK, NUM_CLASSES), jnp.float32),
            pltpu.SemaphoreType.DMA((Q, DEPTH)),
            pltpu.SemaphoreType.DMA((Q, DEPTH)),
        ],
    )(alpha, beta, x)
